# baseline (speedup 1.0000x reference)
import jax
import jax.numpy as jnp
from jax import lax
from jax.experimental import pallas as pl
from jax.experimental.pallas import tpu as pltpu

_BLOCK = 1024


def _match_block(gt_ref, anc_ref, u_ref, mas_ref, match_ref, label_ref):
    gt = gt_ref[...]
    gx0 = gt[:, 0:1]
    gy0 = gt[:, 1:2]
    gx1 = gt[:, 2:3]
    gy1 = gt[:, 3:4]
    anc = anc_ref[...]
    ax0 = anc[0:1, :]
    ay0 = anc[1:2, :]
    ax1 = anc[2:3, :]
    ay1 = anc[3:4, :]
    u = u_ref[...]
    mas = mas_ref[0, 0]

    gt_upper = jnp.maximum(gx1 - gx0, gy1 - gy0) * 2.0
    gt_upper = jnp.where(gt_upper < mas, mas, gt_upper)
    gcx = (gx0 + gx1) / 2.0
    gcy = (gy0 + gy1) / 2.0
    an_size = jnp.maximum(ax1 - ax0, ay1 - ay0) - u
    acx = (ax0 + ax1) / 2.0
    acy = (ay0 + ay1) / 2.0
    uu = u * u

    m = jnp.minimum(gx0 - ax0, gy0 - ay0)
    m = jnp.minimum(m, ax1 - gx1)
    m = jnp.minimum(m, ay1 - gy1)
    m = jnp.minimum(m, gt_upper - an_size)
    dx = gcx - acx
    dy = gcy - acy
    m = jnp.minimum(m, uu - (dx * dx + dy * dy))
    assign = m >= 0.0

    n = gt.shape[0]
    iota = lax.broadcasted_iota(jnp.int32, assign.shape, 0)
    first = jnp.min(jnp.where(assign, iota, n), axis=0, keepdims=True)
    cnt = jnp.sum(jnp.where(assign, 1, 0), axis=0, keepdims=True)
    match_ref[...] = jnp.where(first == n, 0, first)
    label_ref[...] = (cnt == 1).astype(jnp.int8)


def kernel(gt_boxes, anchor_boxes, unit_lengths, min_anchor_size):
    n = gt_boxes.shape[0]
    m = anchor_boxes.shape[0]
    anc_t = anchor_boxes.T
    u2 = unit_lengths.reshape(1, m)
    mas = jnp.asarray(min_anchor_size, jnp.float32).reshape(1, 1)
    matches2, labels2 = pl.pallas_call(
        _match_block,
        grid=(pl.cdiv(m, _BLOCK),),
        in_specs=[
            pl.BlockSpec((n, 4), lambda j: (0, 0)),
            pl.BlockSpec((4, _BLOCK), lambda j: (0, j)),
            pl.BlockSpec((1, _BLOCK), lambda j: (0, j)),
            pl.BlockSpec((1, 1), lambda j: (0, 0)),
        ],
        out_specs=[
            pl.BlockSpec((1, _BLOCK), lambda j: (0, j)),
            pl.BlockSpec((1, _BLOCK), lambda j: (0, j)),
        ],
        out_shape=[
            jax.ShapeDtypeStruct((1, m), jnp.int32),
            jax.ShapeDtypeStruct((1, m), jnp.int8),
        ],
        compiler_params=pltpu.CompilerParams(
            dimension_semantics=("parallel",),
        ),
    )(gt_boxes, anc_t, u2, mas)
    return (matches2.reshape(m), labels2.reshape(m))


# speedup vs baseline: 1.2713x; 1.2713x over previous
import jax
import jax.numpy as jnp
from jax import lax
from jax.experimental import pallas as pl
from jax.experimental.pallas import tpu as pltpu

_BLOCK = 2048


def _match_block(gt_ref, anc_ref, u_ref, mas_ref, match_ref, label_ref):
    gt = gt_ref[...]
    gx0 = gt[:, 0:1]
    gy0 = gt[:, 1:2]
    gx1 = gt[:, 2:3]
    gy1 = gt[:, 3:4]
    anc = anc_ref[...]
    ax0 = anc[0:1, :]
    ay0 = anc[1:2, :]
    ax1 = anc[2:3, :]
    ay1 = anc[3:4, :]
    u = u_ref[...]
    mas = mas_ref[0, 0]

    gt_upper = jnp.maximum(gx1 - gx0, gy1 - gy0) * 2.0
    gt_upper = jnp.where(gt_upper < mas, mas, gt_upper)
    gcx = (gx0 + gx1) / 2.0
    gcy = (gy0 + gy1) / 2.0
    an_size = jnp.maximum(ax1 - ax0, ay1 - ay0) - u
    acx = (ax0 + ax1) / 2.0
    acy = (ay0 + ay1) / 2.0
    uu = u * u

    m = jnp.minimum(gx0 - ax0, gy0 - ay0)
    m = jnp.minimum(m, ax1 - gx1)
    m = jnp.minimum(m, ay1 - gy1)
    m = jnp.minimum(m, gt_upper - an_size)
    dx = gcx - acx
    dy = gcy - acy
    m = jnp.minimum(m, uu - (dx * dx + dy * dy))
    assign = m >= 0.0

    n = gt.shape[0]
    iota = lax.broadcasted_iota(jnp.int32, assign.shape, 0)
    first = jnp.min(jnp.where(assign, iota, n), axis=0, keepdims=True)
    cnt = jnp.sum(jnp.where(assign, 1, 0), axis=0, keepdims=True)
    match_ref[...] = jnp.where(first == n, 0, first)
    label_ref[...] = (cnt == 1).astype(jnp.int8)


def kernel(gt_boxes, anchor_boxes, unit_lengths, min_anchor_size):
    n = gt_boxes.shape[0]
    m = anchor_boxes.shape[0]
    anc_t = anchor_boxes.T
    u2 = unit_lengths.reshape(1, m)
    mas = jnp.asarray(min_anchor_size, jnp.float32).reshape(1, 1)
    matches2, labels2 = pl.pallas_call(
        _match_block,
        grid=(pl.cdiv(m, _BLOCK),),
        in_specs=[
            pl.BlockSpec((n, 4), lambda j: (0, 0)),
            pl.BlockSpec((4, _BLOCK), lambda j: (0, j)),
            pl.BlockSpec((1, _BLOCK), lambda j: (0, j)),
            pl.BlockSpec((1, 1), lambda j: (0, 0)),
        ],
        out_specs=[
            pl.BlockSpec((1, _BLOCK), lambda j: (0, j)),
            pl.BlockSpec((1, _BLOCK), lambda j: (0, j)),
        ],
        out_shape=[
            jax.ShapeDtypeStruct((1, m), jnp.int32),
            jax.ShapeDtypeStruct((1, m), jnp.int8),
        ],
        compiler_params=pltpu.CompilerParams(
            dimension_semantics=("parallel",),
            allow_input_fusion=[True, True, True, True],
        ),
    )(gt_boxes, anc_t, u2, mas)
    return (matches2.reshape(m), labels2.reshape(m))


# speedup vs baseline: 1.2842x; 1.0102x over previous
import jax
import jax.numpy as jnp
from jax import lax
from jax.experimental import pallas as pl
from jax.experimental.pallas import tpu as pltpu

_BLOCK = 2560


def _match_block(gt_ref, anc_ref, u_ref, mas_ref, match_ref, label_ref):
    gt = gt_ref[...]
    gx0 = gt[:, 0:1]
    gy0 = gt[:, 1:2]
    gx1 = gt[:, 2:3]
    gy1 = gt[:, 3:4]
    anc = anc_ref[...]
    ax0 = anc[0:1, :]
    ay0 = anc[1:2, :]
    ax1 = anc[2:3, :]
    ay1 = anc[3:4, :]
    u = u_ref[...]
    mas = mas_ref[0, 0]

    gt_upper = jnp.maximum(gx1 - gx0, gy1 - gy0) * 2.0
    gt_upper = jnp.where(gt_upper < mas, mas, gt_upper)
    gcx = (gx0 + gx1) / 2.0
    gcy = (gy0 + gy1) / 2.0
    an_size = jnp.maximum(ax1 - ax0, ay1 - ay0) - u
    acx = (ax0 + ax1) / 2.0
    acy = (ay0 + ay1) / 2.0
    uu = u * u

    m = jnp.minimum(gx0 - ax0, gy0 - ay0)
    m = jnp.minimum(m, ax1 - gx1)
    m = jnp.minimum(m, ay1 - gy1)
    m = jnp.minimum(m, gt_upper - an_size)
    dx = gcx - acx
    dy = gcy - acy
    m = jnp.minimum(m, uu - (dx * dx + dy * dy))
    assign = m >= 0.0

    n = gt.shape[0]
    iota = lax.broadcasted_iota(jnp.int32, assign.shape, 0)
    first = jnp.min(jnp.where(assign, iota, n), axis=0, keepdims=True)
    cnt = jnp.sum(jnp.where(assign, 1, 0), axis=0, keepdims=True)
    match_ref[...] = jnp.where(first == n, 0, first)
    label_ref[...] = (cnt == 1).astype(jnp.int8)


def kernel(gt_boxes, anchor_boxes, unit_lengths, min_anchor_size):
    n = gt_boxes.shape[0]
    m = anchor_boxes.shape[0]
    anc_t = anchor_boxes.T
    u2 = unit_lengths.reshape(1, m)
    mas = jnp.asarray(min_anchor_size, jnp.float32).reshape(1, 1)
    matches2, labels2 = pl.pallas_call(
        _match_block,
        grid=(pl.cdiv(m, _BLOCK),),
        in_specs=[
            pl.BlockSpec((n, 4), lambda j: (0, 0)),
            pl.BlockSpec((4, _BLOCK), lambda j: (0, j)),
            pl.BlockSpec((1, _BLOCK), lambda j: (0, j)),
            pl.BlockSpec((1, 1), lambda j: (0, 0)),
        ],
        out_specs=[
            pl.BlockSpec((1, _BLOCK), lambda j: (0, j)),
            pl.BlockSpec((1, _BLOCK), lambda j: (0, j)),
        ],
        out_shape=[
            jax.ShapeDtypeStruct((1, m), jnp.int32),
            jax.ShapeDtypeStruct((1, m), jnp.int8),
        ],
        compiler_params=pltpu.CompilerParams(
            dimension_semantics=("parallel",),
            allow_input_fusion=[True, True, True, True],
        ),
    )(gt_boxes, anc_t, u2, mas)
    return (matches2.reshape(m), labels2.reshape(m))
